# baseline (speedup 1.0000x reference)
import functools

import jax
import jax.numpy as jnp
from jax import lax
from jax.experimental import pallas as pl
from jax.experimental.pallas import tpu as pltpu
from jax.experimental.pallas import tpu_sc as plsc

B = 16384
EMBED_K = 64
FEAT_DIM = 512

_NC = 2
_NS = 16
_NW = _NC * _NS
_BPW = B // _NW

_FCHUNK = 64
_NFCHUNK = _BPW // _FCHUNK
_TCHUNK = 128
_NTCHUNK = _BPW // _TCHUNK


def _sc_gather_f(items, F_feat):
    mesh = plsc.VectorSubcoreMesh(core_axis_name="c", subcore_axis_name="s")

    @functools.partial(
        pl.kernel,
        mesh=mesh,
        out_type=jax.ShapeDtypeStruct((B, FEAT_DIM), jnp.float32),
        scratch_types=[
            pltpu.VMEM((_BPW,), jnp.int32),
            pltpu.VMEM((2, _FCHUNK, FEAT_DIM), jnp.float32),
            pltpu.SemaphoreType.DMA,
            pltpu.SemaphoreType.DMA,
        ],
    )
    def k(items_hbm, f_hbm, effe_out, iidx_v, rows_v, sem0, sem1):
        wid = lax.axis_index("s") * _NC + lax.axis_index("c")
        base = wid * _BPW
        pltpu.sync_copy(items_hbm.at[pl.ds(base, _BPW)], iidx_v)
        sems = (sem0, sem1)
        copies = [None, None]
        copies[0] = pltpu.async_copy(
            f_hbm.at[iidx_v.at[pl.ds(0, _FCHUNK)]], rows_v.at[0], sems[0])
        for c in range(_NFCHUNK):
            nxt = c + 1
            if nxt < _NFCHUNK:
                copies[nxt % 2] = pltpu.async_copy(
                    f_hbm.at[iidx_v.at[pl.ds(nxt * _FCHUNK, _FCHUNK)]],
                    rows_v.at[nxt % 2], sems[nxt % 2])
            copies[c % 2].wait()
            pltpu.sync_copy(rows_v.at[c % 2],
                            effe_out.at[pl.ds(base + c * _FCHUNK, _FCHUNK)])

    return k(items, F_feat)


def _sc_gather_tu(upairs, Tu_pairs):
    mesh = plsc.VectorSubcoreMesh(core_axis_name="c", subcore_axis_name="s")

    @functools.partial(
        pl.kernel,
        mesh=mesh,
        out_type=jax.ShapeDtypeStruct((B, 2 * EMBED_K), jnp.float32),
        scratch_types=[
            pltpu.VMEM((_BPW,), jnp.int32),
            pltpu.VMEM((2, _TCHUNK, 2 * EMBED_K), jnp.float32),
            pltpu.SemaphoreType.DMA,
            pltpu.SemaphoreType.DMA,
        ],
    )
    def k(upairs_hbm, tu_hbm, theta_out, uidx_v, rows_v, sem0, sem1):
        wid = lax.axis_index("s") * _NC + lax.axis_index("c")
        base = wid * _BPW
        pltpu.sync_copy(upairs_hbm.at[pl.ds(base, _BPW)], uidx_v)
        sems = (sem0, sem1)
        copies = [None, None]
        copies[0] = pltpu.async_copy(
            tu_hbm.at[uidx_v.at[pl.ds(0, _TCHUNK)]], rows_v.at[0], sems[0])
        for c in range(_NTCHUNK):
            nxt = c + 1
            if nxt < _NTCHUNK:
                copies[nxt % 2] = pltpu.async_copy(
                    tu_hbm.at[uidx_v.at[pl.ds(nxt * _TCHUNK, _TCHUNK)]],
                    rows_v.at[nxt % 2], sems[nxt % 2])
            copies[c % 2].wait()
            pltpu.sync_copy(rows_v.at[c % 2],
                            theta_out.at[pl.ds(base + c * _TCHUNK, _TCHUNK)])

    return k(upairs, Tu_pairs)


_TBLK = 1024


def _tc_dense_body(gu_ref, gi_ref, theta2_ref, upar_ref, effe_ref, w_ref,
                   b_ref, xui_ref, theta_ref, proj_ref):
    proj = jnp.dot(effe_ref[...], w_ref[...],
                   preferred_element_type=jnp.float32) + b_ref[...]
    nrm = jnp.sqrt(jnp.sum(proj * proj, axis=1, keepdims=True))
    proj_i = proj / jnp.maximum(nrm, 1e-12)
    par = (upar_ref[...] & 1)[:, None]
    theta = jnp.where(par == 1, theta2_ref[:, EMBED_K:], theta2_ref[:, :EMBED_K])
    xui_ref[...] = (jnp.sum(gu_ref[...] * gi_ref[...], axis=1)
                    + jnp.sum(theta * proj_i, axis=1))
    theta_ref[...] = theta
    proj_ref[...] = proj_i


def _tc_dense(gu, gi, theta2, users, effe_i, proj_W, proj_b):
    return pl.pallas_call(
        _tc_dense_body,
        grid=(B // _TBLK,),
        in_specs=[
            pl.BlockSpec((_TBLK, EMBED_K), lambda i: (i, 0)),
            pl.BlockSpec((_TBLK, EMBED_K), lambda i: (i, 0)),
            pl.BlockSpec((_TBLK, 2 * EMBED_K), lambda i: (i, 0)),
            pl.BlockSpec((_TBLK,), lambda i: (i,)),
            pl.BlockSpec((_TBLK, FEAT_DIM), lambda i: (i, 0)),
            pl.BlockSpec((FEAT_DIM, EMBED_K), lambda i: (0, 0)),
            pl.BlockSpec((1, EMBED_K), lambda i: (0, 0)),
        ],
        out_specs=[
            pl.BlockSpec((_TBLK,), lambda i: (i,)),
            pl.BlockSpec((_TBLK, EMBED_K), lambda i: (i, 0)),
            pl.BlockSpec((_TBLK, EMBED_K), lambda i: (i, 0)),
        ],
        out_shape=[
            jax.ShapeDtypeStruct((B,), jnp.float32),
            jax.ShapeDtypeStruct((B, EMBED_K), jnp.float32),
            jax.ShapeDtypeStruct((B, EMBED_K), jnp.float32),
        ],
    )(gu, gi, theta2, users, effe_i, proj_W, proj_b.reshape(1, EMBED_K))


def kernel(gu, gi, users, items, Tu_weight, F_feat, proj_W, proj_b):
    effe_i = _sc_gather_f(items, F_feat)
    Tu_pairs = Tu_weight.reshape(Tu_weight.shape[0] // 2, 2 * EMBED_K)
    theta2 = _sc_gather_tu(lax.shift_right_logical(users, 1), Tu_pairs)
    xui, theta_u, proj_i = _tc_dense(gu, gi, theta2, users, effe_i,
                                     proj_W, proj_b)
    return (xui, gu, gi, theta_u, proj_i)


# speedup vs baseline: 1.0152x; 1.0152x over previous
import functools

import jax
import jax.numpy as jnp
from jax import lax
from jax.experimental import pallas as pl
from jax.experimental.pallas import tpu as pltpu
from jax.experimental.pallas import tpu_sc as plsc

B = 16384
EMBED_K = 64
FEAT_DIM = 512

_NC = 2
_NS = 16
_NW = _NC * _NS
_BPW = B // _NW

_FCHUNK = 64
_NFCHUNK = _BPW // _FCHUNK
_TCHUNK = 128
_NTCHUNK = _BPW // _TCHUNK


def _sc_gather_f(items, F_feat):
    mesh = plsc.VectorSubcoreMesh(core_axis_name="c", subcore_axis_name="s")

    @functools.partial(
        pl.kernel,
        mesh=mesh,
        out_type=jax.ShapeDtypeStruct((B, FEAT_DIM), jnp.float32),
        scratch_types=[
            pltpu.VMEM((_BPW,), jnp.int32),
            pltpu.VMEM((2, _FCHUNK, FEAT_DIM), jnp.float32),
            pltpu.SemaphoreType.DMA,
            pltpu.SemaphoreType.DMA,
        ],
    )
    def k(items_hbm, f_hbm, effe_out, iidx_v, rows_v, sem0, sem1):
        wid = lax.axis_index("s") * _NC + lax.axis_index("c")
        base = wid * _BPW
        pltpu.sync_copy(items_hbm.at[pl.ds(base, _BPW)], iidx_v)
        sems = (sem0, sem1)
        copies = [None, None]
        copies[0] = pltpu.async_copy(
            f_hbm.at[iidx_v.at[pl.ds(0, _FCHUNK)]], rows_v.at[0], sems[0])
        for c in range(_NFCHUNK):
            nxt = c + 1
            if nxt < _NFCHUNK:
                copies[nxt % 2] = pltpu.async_copy(
                    f_hbm.at[iidx_v.at[pl.ds(nxt * _FCHUNK, _FCHUNK)]],
                    rows_v.at[nxt % 2], sems[nxt % 2])
            copies[c % 2].wait()
            pltpu.sync_copy(rows_v.at[c % 2],
                            effe_out.at[pl.ds(base + c * _FCHUNK, _FCHUNK)])

    return k(items, F_feat)


def _sc_gather_tu(upairs, Tu_pairs):
    mesh = plsc.VectorSubcoreMesh(core_axis_name="c", subcore_axis_name="s")

    @functools.partial(
        pl.kernel,
        mesh=mesh,
        out_type=jax.ShapeDtypeStruct((B, 2 * EMBED_K), jnp.float32),
        scratch_types=[
            pltpu.VMEM((_BPW,), jnp.int32),
            pltpu.VMEM((2, _TCHUNK, 2 * EMBED_K), jnp.float32),
            pltpu.SemaphoreType.DMA,
            pltpu.SemaphoreType.DMA,
        ],
    )
    def k(upairs_hbm, tu_hbm, theta_out, uidx_v, rows_v, sem0, sem1):
        wid = lax.axis_index("s") * _NC + lax.axis_index("c")
        base = wid * _BPW
        pltpu.sync_copy(upairs_hbm.at[pl.ds(base, _BPW)], uidx_v)
        sems = (sem0, sem1)
        copies = [None, None]
        copies[0] = pltpu.async_copy(
            tu_hbm.at[uidx_v.at[pl.ds(0, _TCHUNK)]], rows_v.at[0], sems[0])
        for c in range(_NTCHUNK):
            nxt = c + 1
            if nxt < _NTCHUNK:
                copies[nxt % 2] = pltpu.async_copy(
                    tu_hbm.at[uidx_v.at[pl.ds(nxt * _TCHUNK, _TCHUNK)]],
                    rows_v.at[nxt % 2], sems[nxt % 2])
            copies[c % 2].wait()
            pltpu.sync_copy(rows_v.at[c % 2],
                            theta_out.at[pl.ds(base + c * _TCHUNK, _TCHUNK)])

    return k(upairs, Tu_pairs)


_TBLK = 2048


def _tc_dense_body(gu_ref, gi_ref, theta2_ref, upar_ref, effe_ref, w_ref,
                   b_ref, xui_ref, theta_ref, proj_ref, gou_ref, goi_ref):
    proj = jnp.dot(effe_ref[...], w_ref[...],
                   preferred_element_type=jnp.float32) + b_ref[...]
    ones = jnp.ones((EMBED_K, 1), jnp.float32)
    s = jnp.dot(proj * proj, ones, preferred_element_type=jnp.float32)
    inv = jnp.where(s > 1e-24, lax.rsqrt(s), 1e12)
    proj_i = proj * inv
    par = (upar_ref[...] & 1)[:, None]
    theta = jnp.where(par == 1, theta2_ref[:, EMBED_K:], theta2_ref[:, :EMBED_K])
    gu = gu_ref[...]
    gi = gi_ref[...]
    xui_ref[...] = jnp.dot(gu * gi + theta * proj_i, ones,
                           preferred_element_type=jnp.float32)
    theta_ref[...] = theta
    proj_ref[...] = proj_i
    gou_ref[...] = gu
    goi_ref[...] = gi


def _tc_dense(gu, gi, theta2, users, effe_i, proj_W, proj_b):
    return pl.pallas_call(
        _tc_dense_body,
        grid=(B // _TBLK,),
        in_specs=[
            pl.BlockSpec((_TBLK, EMBED_K), lambda i: (i, 0)),
            pl.BlockSpec((_TBLK, EMBED_K), lambda i: (i, 0)),
            pl.BlockSpec((_TBLK, 2 * EMBED_K), lambda i: (i, 0)),
            pl.BlockSpec((_TBLK,), lambda i: (i,)),
            pl.BlockSpec((_TBLK, FEAT_DIM), lambda i: (i, 0)),
            pl.BlockSpec((FEAT_DIM, EMBED_K), lambda i: (0, 0)),
            pl.BlockSpec((1, EMBED_K), lambda i: (0, 0)),
        ],
        out_specs=[
            pl.BlockSpec((_TBLK, 1), lambda i: (i, 0)),
            pl.BlockSpec((_TBLK, EMBED_K), lambda i: (i, 0)),
            pl.BlockSpec((_TBLK, EMBED_K), lambda i: (i, 0)),
            pl.BlockSpec((_TBLK, EMBED_K), lambda i: (i, 0)),
            pl.BlockSpec((_TBLK, EMBED_K), lambda i: (i, 0)),
        ],
        out_shape=[
            jax.ShapeDtypeStruct((B, 1), jnp.float32),
            jax.ShapeDtypeStruct((B, EMBED_K), jnp.float32),
            jax.ShapeDtypeStruct((B, EMBED_K), jnp.float32),
            jax.ShapeDtypeStruct((B, EMBED_K), jnp.float32),
            jax.ShapeDtypeStruct((B, EMBED_K), jnp.float32),
        ],
    )(gu, gi, theta2, users, effe_i, proj_W, proj_b.reshape(1, EMBED_K))


def kernel(gu, gi, users, items, Tu_weight, F_feat, proj_W, proj_b):
    effe_i = _sc_gather_f(items, F_feat)
    Tu_pairs = Tu_weight.reshape(Tu_weight.shape[0] // 2, 2 * EMBED_K)
    theta2 = _sc_gather_tu(lax.shift_right_logical(users, 1), Tu_pairs)
    xui2, theta_u, proj_i, gamma_u, gamma_i = _tc_dense(
        gu, gi, theta2, users, effe_i, proj_W, proj_b)
    return (jnp.reshape(xui2, (B,)), gamma_u, gamma_i, theta_u, proj_i)


# speedup vs baseline: 1.4122x; 1.3911x over previous
import functools

import jax
import jax.numpy as jnp
from jax import lax
from jax.experimental import pallas as pl
from jax.experimental.pallas import tpu as pltpu
from jax.experimental.pallas import tpu_sc as plsc

B = 16384
EMBED_K = 64
FEAT_DIM = 512

_NC = 2
_NS = 16
_NW = _NC * _NS
_BPW = B // _NW

_FCHUNK = 64
_NFCHUNK = _BPW // _FCHUNK
_TCHUNK = 128
_NTCHUNK = _BPW // _TCHUNK


def _sc_gather_f(items, F_feat):
    mesh = plsc.VectorSubcoreMesh(core_axis_name="c", subcore_axis_name="s")

    @functools.partial(
        pl.kernel,
        mesh=mesh,
        out_type=jax.ShapeDtypeStruct((B, FEAT_DIM), jnp.float32),
        scratch_types=[
            pltpu.VMEM((_BPW,), jnp.int32),
            pltpu.VMEM((2, _FCHUNK, FEAT_DIM), jnp.float32),
            pltpu.SemaphoreType.DMA,
            pltpu.SemaphoreType.DMA,
        ],
    )
    def k(items_hbm, f_hbm, effe_out, iidx_v, rows_v, sem0, sem1):
        wid = lax.axis_index("s") * _NC + lax.axis_index("c")
        base = wid * _BPW
        pltpu.sync_copy(items_hbm.at[pl.ds(base, _BPW)], iidx_v)
        sems = (sem0, sem1)
        copies = [None, None]
        copies[0] = pltpu.async_copy(
            f_hbm.at[iidx_v.at[pl.ds(0, _FCHUNK)]], rows_v.at[0], sems[0])
        for c in range(_NFCHUNK):
            nxt = c + 1
            if nxt < _NFCHUNK:
                copies[nxt % 2] = pltpu.async_copy(
                    f_hbm.at[iidx_v.at[pl.ds(nxt * _FCHUNK, _FCHUNK)]],
                    rows_v.at[nxt % 2], sems[nxt % 2])
            copies[c % 2].wait()
            pltpu.sync_copy(rows_v.at[c % 2],
                            effe_out.at[pl.ds(base + c * _FCHUNK, _FCHUNK)])

    return k(items, F_feat)


def _sc_gather_tu(upairs, Tu_pairs):
    mesh = plsc.VectorSubcoreMesh(core_axis_name="c", subcore_axis_name="s")

    @functools.partial(
        pl.kernel,
        mesh=mesh,
        out_type=jax.ShapeDtypeStruct((B, 2 * EMBED_K), jnp.float32),
        scratch_types=[
            pltpu.VMEM((_BPW,), jnp.int32),
            pltpu.VMEM((2, _TCHUNK, 2 * EMBED_K), jnp.float32),
            pltpu.SemaphoreType.DMA,
            pltpu.SemaphoreType.DMA,
        ],
    )
    def k(upairs_hbm, tu_hbm, theta_out, uidx_v, rows_v, sem0, sem1):
        wid = lax.axis_index("s") * _NC + lax.axis_index("c")
        base = wid * _BPW
        pltpu.sync_copy(upairs_hbm.at[pl.ds(base, _BPW)], uidx_v)
        sems = (sem0, sem1)
        copies = [None, None]
        copies[0] = pltpu.async_copy(
            tu_hbm.at[uidx_v.at[pl.ds(0, _TCHUNK)]], rows_v.at[0], sems[0])
        for c in range(_NTCHUNK):
            nxt = c + 1
            if nxt < _NTCHUNK:
                copies[nxt % 2] = pltpu.async_copy(
                    tu_hbm.at[uidx_v.at[pl.ds(nxt * _TCHUNK, _TCHUNK)]],
                    rows_v.at[nxt % 2], sems[nxt % 2])
            copies[c % 2].wait()
            pltpu.sync_copy(rows_v.at[c % 2],
                            theta_out.at[pl.ds(base + c * _TCHUNK, _TCHUNK)])

    return k(upairs, Tu_pairs)


_TBLK = 2048


def _tc_dense_body(guT_ref, giT_ref, theta2_ref, upar_ref, effe_ref, w_ref,
                   b_ref, xui_ref, thetaT_ref, projT_ref, gouT_ref, goiT_ref):
    proj = jnp.dot(effe_ref[...], w_ref[...],
                   preferred_element_type=jnp.float32)
    par = (upar_ref[...] & 1)[:, None]
    theta = jnp.where(par == 1, theta2_ref[:, EMBED_K:], theta2_ref[:, :EMBED_K])
    projT = jnp.transpose(proj) + b_ref[...]
    thetaT = jnp.transpose(theta)
    ones = jnp.ones((1, EMBED_K), jnp.float32)
    sT = jnp.dot(ones, projT * projT, preferred_element_type=jnp.float32)
    invT = jnp.where(sT > 1e-24, lax.rsqrt(sT), 1e12)
    proj_iT = projT * invT
    guT = guT_ref[...]
    giT = giT_ref[...]
    xui = jnp.dot(ones, guT * giT + thetaT * proj_iT,
                  preferred_element_type=jnp.float32)
    xui_ref[...] = xui[None]
    thetaT_ref[...] = thetaT
    projT_ref[...] = proj_iT
    gouT_ref[...] = guT
    goiT_ref[...] = giT


def _tc_dense(guT, giT, theta2, users, effe_i, proj_W, proj_b):
    return pl.pallas_call(
        _tc_dense_body,
        grid=(B // _TBLK,),
        in_specs=[
            pl.BlockSpec((EMBED_K, _TBLK), lambda i: (0, i)),
            pl.BlockSpec((EMBED_K, _TBLK), lambda i: (0, i)),
            pl.BlockSpec((_TBLK, 2 * EMBED_K), lambda i: (i, 0)),
            pl.BlockSpec((_TBLK,), lambda i: (i,)),
            pl.BlockSpec((_TBLK, FEAT_DIM), lambda i: (i, 0)),
            pl.BlockSpec((FEAT_DIM, EMBED_K), lambda i: (0, 0)),
            pl.BlockSpec((EMBED_K, 1), lambda i: (0, 0)),
        ],
        out_specs=[
            pl.BlockSpec((1, 1, _TBLK), lambda i: (i, 0, 0)),
            pl.BlockSpec((EMBED_K, _TBLK), lambda i: (0, i)),
            pl.BlockSpec((EMBED_K, _TBLK), lambda i: (0, i)),
            pl.BlockSpec((EMBED_K, _TBLK), lambda i: (0, i)),
            pl.BlockSpec((EMBED_K, _TBLK), lambda i: (0, i)),
        ],
        out_shape=[
            jax.ShapeDtypeStruct((B // _TBLK, 1, _TBLK), jnp.float32),
            jax.ShapeDtypeStruct((EMBED_K, B), jnp.float32),
            jax.ShapeDtypeStruct((EMBED_K, B), jnp.float32),
            jax.ShapeDtypeStruct((EMBED_K, B), jnp.float32),
            jax.ShapeDtypeStruct((EMBED_K, B), jnp.float32),
        ],
    )(guT, giT, theta2, users, effe_i, proj_W,
      jnp.reshape(proj_b, (EMBED_K, 1)))


def kernel(gu, gi, users, items, Tu_weight, F_feat, proj_W, proj_b):
    effe_i = _sc_gather_f(items, F_feat)
    Tu_pairs = Tu_weight.reshape(Tu_weight.shape[0] // 2, 2 * EMBED_K)
    theta2 = _sc_gather_tu(lax.shift_right_logical(users, 1), Tu_pairs)
    xui2, thetaT, projT, gammaT_u, gammaT_i = _tc_dense(
        jnp.transpose(gu), jnp.transpose(gi), theta2, users, effe_i,
        proj_W, proj_b)
    return (jnp.reshape(xui2, (B,)), jnp.transpose(gammaT_u),
            jnp.transpose(gammaT_i), jnp.transpose(thetaT),
            jnp.transpose(projT))


# speedup vs baseline: 1.6586x; 1.1745x over previous
import functools

import jax
import jax.numpy as jnp
from jax import lax
from jax.experimental import pallas as pl
from jax.experimental.pallas import tpu as pltpu
from jax.experimental.pallas import tpu_sc as plsc

B = 16384
EMBED_K = 64
FEAT_DIM = 512

_NC = 2
_NS = 16
_NW = _NC * _NS
_BPW = B // _NW

_FCHUNK = 64
_NFCHUNK = _BPW // _FCHUNK
_TCHUNK = 128
_NTCHUNK = _BPW // _TCHUNK


def _sc_gather_f(items, F_feat):
    mesh = plsc.VectorSubcoreMesh(core_axis_name="c", subcore_axis_name="s")

    @functools.partial(
        pl.kernel,
        mesh=mesh,
        out_type=jax.ShapeDtypeStruct((B, FEAT_DIM), jnp.float32),
        scratch_types=[
            pltpu.VMEM((_BPW,), jnp.int32),
            pltpu.VMEM((2, _FCHUNK, FEAT_DIM), jnp.float32),
            pltpu.SemaphoreType.DMA,
            pltpu.SemaphoreType.DMA,
        ],
    )
    def k(items_hbm, f_hbm, effe_out, iidx_v, rows_v, sem0, sem1):
        wid = lax.axis_index("s") * _NC + lax.axis_index("c")
        base = wid * _BPW
        pltpu.sync_copy(items_hbm.at[pl.ds(base, _BPW)], iidx_v)
        sems = (sem0, sem1)
        copies = [None, None]
        copies[0] = pltpu.async_copy(
            f_hbm.at[iidx_v.at[pl.ds(0, _FCHUNK)]], rows_v.at[0], sems[0])
        for c in range(_NFCHUNK):
            nxt = c + 1
            if nxt < _NFCHUNK:
                copies[nxt % 2] = pltpu.async_copy(
                    f_hbm.at[iidx_v.at[pl.ds(nxt * _FCHUNK, _FCHUNK)]],
                    rows_v.at[nxt % 2], sems[nxt % 2])
            copies[c % 2].wait()
            pltpu.sync_copy(rows_v.at[c % 2],
                            effe_out.at[pl.ds(base + c * _FCHUNK, _FCHUNK)])

    return k(items, F_feat)


def _sc_gather_tu(upairs, Tu_pairs):
    mesh = plsc.VectorSubcoreMesh(core_axis_name="c", subcore_axis_name="s")

    @functools.partial(
        pl.kernel,
        mesh=mesh,
        out_type=jax.ShapeDtypeStruct((B, 2 * EMBED_K), jnp.float32),
        scratch_types=[
            pltpu.VMEM((_BPW,), jnp.int32),
            pltpu.VMEM((2, _TCHUNK, 2 * EMBED_K), jnp.float32),
            pltpu.SemaphoreType.DMA,
            pltpu.SemaphoreType.DMA,
        ],
    )
    def k(upairs_hbm, tu_hbm, theta_out, uidx_v, rows_v, sem0, sem1):
        wid = lax.axis_index("s") * _NC + lax.axis_index("c")
        base = wid * _BPW
        pltpu.sync_copy(upairs_hbm.at[pl.ds(base, _BPW)], uidx_v)
        sems = (sem0, sem1)
        copies = [None, None]
        copies[0] = pltpu.async_copy(
            tu_hbm.at[uidx_v.at[pl.ds(0, _TCHUNK)]], rows_v.at[0], sems[0])
        for c in range(_NTCHUNK):
            nxt = c + 1
            if nxt < _NTCHUNK:
                copies[nxt % 2] = pltpu.async_copy(
                    tu_hbm.at[uidx_v.at[pl.ds(nxt * _TCHUNK, _TCHUNK)]],
                    rows_v.at[nxt % 2], sems[nxt % 2])
            copies[c % 2].wait()
            pltpu.sync_copy(rows_v.at[c % 2],
                            theta_out.at[pl.ds(base + c * _TCHUNK, _TCHUNK)])

    return k(upairs, Tu_pairs)


_PLANES = 8192


def _tc_pairs(TuT):
    NU = TuT.shape[1]
    nb = (NU + _PLANES - 1) // _PLANES
    half = _PLANES // 2

    def body(x_ref, o_ref):
        y = jnp.transpose(x_ref[...])
        o_ref[...] = jnp.concatenate(
            [lax.slice(y, (0, 0), (half, EMBED_K)),
             lax.slice(y, (half, 0), (_PLANES, EMBED_K))], axis=1)

    return pl.pallas_call(
        body,
        grid=(nb,),
        in_specs=[pl.BlockSpec((EMBED_K, _PLANES), lambda i: (0, i))],
        out_specs=pl.BlockSpec((half, 2 * EMBED_K), lambda i: (i, 0)),
        out_shape=jax.ShapeDtypeStruct((nb * half, 2 * EMBED_K), jnp.float32),
    )(TuT)


_TBLK = 2048


def _tc_dense_body(guT_ref, giT_ref, theta2_ref, upar_ref, effe_ref, w_ref,
                   b_ref, xui_ref, thetaT_ref, projT_ref, gouT_ref, goiT_ref):
    proj = jnp.dot(effe_ref[...], w_ref[...],
                   preferred_element_type=jnp.float32)
    par = (lax.shift_right_logical(upar_ref[...], 12) & 1)[:, None]
    theta = jnp.where(par == 1, theta2_ref[:, EMBED_K:], theta2_ref[:, :EMBED_K])
    projT = jnp.transpose(proj) + b_ref[...]
    thetaT = jnp.transpose(theta)
    ones = jnp.ones((1, EMBED_K), jnp.float32)
    sT = jnp.dot(ones, projT * projT, preferred_element_type=jnp.float32)
    invT = jnp.where(sT > 1e-24, lax.rsqrt(sT), 1e12)
    proj_iT = projT * invT
    guT = guT_ref[...]
    giT = giT_ref[...]
    xui = jnp.dot(ones, guT * giT + thetaT * proj_iT,
                  preferred_element_type=jnp.float32)
    xui_ref[...] = xui[None]
    thetaT_ref[...] = thetaT
    projT_ref[...] = proj_iT
    gouT_ref[...] = guT
    goiT_ref[...] = giT


def _tc_dense(guT, giT, theta2, users, effe_i, proj_W, proj_b):
    return pl.pallas_call(
        _tc_dense_body,
        grid=(B // _TBLK,),
        in_specs=[
            pl.BlockSpec((EMBED_K, _TBLK), lambda i: (0, i)),
            pl.BlockSpec((EMBED_K, _TBLK), lambda i: (0, i)),
            pl.BlockSpec((_TBLK, 2 * EMBED_K), lambda i: (i, 0)),
            pl.BlockSpec((_TBLK,), lambda i: (i,)),
            pl.BlockSpec((_TBLK, FEAT_DIM), lambda i: (i, 0)),
            pl.BlockSpec((FEAT_DIM, EMBED_K), lambda i: (0, 0)),
            pl.BlockSpec((EMBED_K, 1), lambda i: (0, 0)),
        ],
        out_specs=[
            pl.BlockSpec((1, 1, _TBLK), lambda i: (i, 0, 0)),
            pl.BlockSpec((EMBED_K, _TBLK), lambda i: (0, i)),
            pl.BlockSpec((EMBED_K, _TBLK), lambda i: (0, i)),
            pl.BlockSpec((EMBED_K, _TBLK), lambda i: (0, i)),
            pl.BlockSpec((EMBED_K, _TBLK), lambda i: (0, i)),
        ],
        out_shape=[
            jax.ShapeDtypeStruct((B // _TBLK, 1, _TBLK), jnp.float32),
            jax.ShapeDtypeStruct((EMBED_K, B), jnp.float32),
            jax.ShapeDtypeStruct((EMBED_K, B), jnp.float32),
            jax.ShapeDtypeStruct((EMBED_K, B), jnp.float32),
            jax.ShapeDtypeStruct((EMBED_K, B), jnp.float32),
        ],
    )(guT, giT, theta2, users, effe_i, proj_W,
      jnp.reshape(proj_b, (EMBED_K, 1)))


def kernel(gu, gi, users, items, Tu_weight, F_feat, proj_W, proj_b):
    effe_i = _sc_gather_f(items, F_feat)
    Tu_pairs = _tc_pairs(jnp.transpose(Tu_weight))
    zidx = (lax.shift_right_logical(users, 13) * (_PLANES // 2)
            + (users & (_PLANES // 2 - 1)))
    theta2 = _sc_gather_tu(zidx, Tu_pairs)
    xui2, thetaT, projT, gammaT_u, gammaT_i = _tc_dense(
        jnp.transpose(gu), jnp.transpose(gi), theta2, users, effe_i,
        proj_W, proj_b)
    return (jnp.reshape(xui2, (B,)), jnp.transpose(gammaT_u),
            jnp.transpose(gammaT_i), jnp.transpose(thetaT),
            jnp.transpose(projT))


# speedup vs baseline: 1.8952x; 1.1426x over previous
import functools

import jax
import jax.numpy as jnp
from jax import lax
from jax.experimental import pallas as pl
from jax.experimental.pallas import tpu as pltpu
from jax.experimental.pallas import tpu_sc as plsc

B = 16384
EMBED_K = 64
FEAT_DIM = 512

_NC = 2
_NS = 16
_NW = _NC * _NS
_BPW = B // _NW

_FCHUNK = 64
_NFCHUNK = _BPW // _FCHUNK
_TCHUNK = 128
_NTCHUNK = _BPW // _TCHUNK


def _sc_gather_f(items, F_feat):
    mesh = plsc.VectorSubcoreMesh(core_axis_name="c", subcore_axis_name="s")

    @functools.partial(
        pl.kernel,
        mesh=mesh,
        out_type=jax.ShapeDtypeStruct((B, FEAT_DIM), jnp.float32),
        scratch_types=[
            pltpu.VMEM((_BPW,), jnp.int32),
            pltpu.VMEM((2, _FCHUNK, FEAT_DIM), jnp.float32),
            pltpu.SemaphoreType.DMA,
            pltpu.SemaphoreType.DMA,
        ],
    )
    def k(items_hbm, f_hbm, effe_out, iidx_v, rows_v, sem0, sem1):
        wid = lax.axis_index("s") * _NC + lax.axis_index("c")
        base = wid * _BPW
        pltpu.sync_copy(items_hbm.at[pl.ds(base, _BPW)], iidx_v)
        sems = (sem0, sem1)
        copies = [None, None]
        copies[0] = pltpu.async_copy(
            f_hbm.at[iidx_v.at[pl.ds(0, _FCHUNK)]], rows_v.at[0], sems[0])
        for c in range(_NFCHUNK):
            nxt = c + 1
            if nxt < _NFCHUNK:
                copies[nxt % 2] = pltpu.async_copy(
                    f_hbm.at[iidx_v.at[pl.ds(nxt * _FCHUNK, _FCHUNK)]],
                    rows_v.at[nxt % 2], sems[nxt % 2])
            copies[c % 2].wait()
            pltpu.sync_copy(rows_v.at[c % 2],
                            effe_out.at[pl.ds(base + c * _FCHUNK, _FCHUNK)])

    return k(items, F_feat)


def _sc_gather_tu(upairs, Tu_pairs):
    mesh = plsc.VectorSubcoreMesh(core_axis_name="c", subcore_axis_name="s")

    @functools.partial(
        pl.kernel,
        mesh=mesh,
        out_type=jax.ShapeDtypeStruct((B, 2 * EMBED_K), jnp.float32),
        scratch_types=[
            pltpu.VMEM((_BPW,), jnp.int32),
            pltpu.VMEM((2, _TCHUNK, 2 * EMBED_K), jnp.float32),
            pltpu.SemaphoreType.DMA,
            pltpu.SemaphoreType.DMA,
        ],
    )
    def k(upairs_hbm, tu_hbm, theta_out, uidx_v, rows_v, sem0, sem1):
        wid = lax.axis_index("s") * _NC + lax.axis_index("c")
        base = wid * _BPW
        pltpu.sync_copy(upairs_hbm.at[pl.ds(base, _BPW)], uidx_v)
        sems = (sem0, sem1)
        copies = [None, None]
        copies[0] = pltpu.async_copy(
            tu_hbm.at[uidx_v.at[pl.ds(0, _TCHUNK)]], rows_v.at[0], sems[0])
        for c in range(_NTCHUNK):
            nxt = c + 1
            if nxt < _NTCHUNK:
                copies[nxt % 2] = pltpu.async_copy(
                    tu_hbm.at[uidx_v.at[pl.ds(nxt * _TCHUNK, _TCHUNK)]],
                    rows_v.at[nxt % 2], sems[nxt % 2])
            copies[c % 2].wait()
            pltpu.sync_copy(rows_v.at[c % 2],
                            theta_out.at[pl.ds(base + c * _TCHUNK, _TCHUNK)])

    return k(upairs, Tu_pairs)


_PLANES = 8192


def _tc_pairs(TuT):
    NU = TuT.shape[1]
    nb = (NU + _PLANES - 1) // _PLANES
    half = _PLANES // 2

    def body(x_ref, o_ref):
        y = jnp.transpose(x_ref[...])
        o_ref[...] = jnp.concatenate(
            [lax.slice(y, (0, 0), (half, EMBED_K)),
             lax.slice(y, (half, 0), (_PLANES, EMBED_K))], axis=1)

    return pl.pallas_call(
        body,
        grid=(nb,),
        in_specs=[pl.BlockSpec((EMBED_K, _PLANES), lambda i: (0, i))],
        out_specs=pl.BlockSpec((half, 2 * EMBED_K), lambda i: (i, 0)),
        out_shape=jax.ShapeDtypeStruct((nb * half, 2 * EMBED_K), jnp.float32),
    )(TuT)


_TBLK = 2048


def _tc_gammas(guT, giT):
    def body(a_ref, b_ref, oa_ref, ob_ref):
        oa_ref[...] = a_ref[...]
        ob_ref[...] = b_ref[...]

    return pl.pallas_call(
        body,
        grid=(B // _TBLK,),
        in_specs=[
            pl.BlockSpec((EMBED_K, _TBLK), lambda i: (0, i)),
            pl.BlockSpec((EMBED_K, _TBLK), lambda i: (0, i)),
        ],
        out_specs=[
            pl.BlockSpec((EMBED_K, _TBLK), lambda i: (0, i)),
            pl.BlockSpec((EMBED_K, _TBLK), lambda i: (0, i)),
        ],
        out_shape=[
            jax.ShapeDtypeStruct((EMBED_K, B), jnp.float32),
            jax.ShapeDtypeStruct((EMBED_K, B), jnp.float32),
        ],
    )(guT, giT)


def _tc_dense_body(guT_ref, giT_ref, theta2_ref, upar_ref, effe_ref, w_ref,
                   b_ref, xui_ref, thetaT_ref, projT_ref):
    proj = jnp.dot(effe_ref[...], w_ref[...],
                   preferred_element_type=jnp.float32)
    par = (lax.shift_right_logical(upar_ref[...], 12) & 1)[:, None]
    theta = jnp.where(par == 1, theta2_ref[:, EMBED_K:], theta2_ref[:, :EMBED_K])
    projT = jnp.transpose(proj) + b_ref[...]
    thetaT = jnp.transpose(theta)
    ones = jnp.ones((1, EMBED_K), jnp.float32)
    sT = jnp.dot(ones, projT * projT, preferred_element_type=jnp.float32)
    invT = jnp.where(sT > 1e-24, lax.rsqrt(sT), 1e12)
    proj_iT = projT * invT
    guT = guT_ref[...]
    giT = giT_ref[...]
    xui = jnp.dot(ones, guT * giT + thetaT * proj_iT,
                  preferred_element_type=jnp.float32)
    xui_ref[...] = xui[None]
    thetaT_ref[...] = thetaT
    projT_ref[...] = proj_iT


def _tc_dense(guT, giT, theta2, users, effe_i, proj_W, proj_b):
    return pl.pallas_call(
        _tc_dense_body,
        grid=(B // _TBLK,),
        in_specs=[
            pl.BlockSpec((EMBED_K, _TBLK), lambda i: (0, i)),
            pl.BlockSpec((EMBED_K, _TBLK), lambda i: (0, i)),
            pl.BlockSpec((_TBLK, 2 * EMBED_K), lambda i: (i, 0)),
            pl.BlockSpec((_TBLK,), lambda i: (i,)),
            pl.BlockSpec((_TBLK, FEAT_DIM), lambda i: (i, 0)),
            pl.BlockSpec((FEAT_DIM, EMBED_K), lambda i: (0, 0)),
            pl.BlockSpec((EMBED_K, 1), lambda i: (0, 0)),
        ],
        out_specs=[
            pl.BlockSpec((1, 1, _TBLK), lambda i: (i, 0, 0)),
            pl.BlockSpec((EMBED_K, _TBLK), lambda i: (0, i)),
            pl.BlockSpec((EMBED_K, _TBLK), lambda i: (0, i)),
        ],
        out_shape=[
            jax.ShapeDtypeStruct((B // _TBLK, 1, _TBLK), jnp.float32),
            jax.ShapeDtypeStruct((EMBED_K, B), jnp.float32),
            jax.ShapeDtypeStruct((EMBED_K, B), jnp.float32),
        ],
    )(guT, giT, theta2, users, effe_i, proj_W,
      jnp.reshape(proj_b, (EMBED_K, 1)))


def kernel(gu, gi, users, items, Tu_weight, F_feat, proj_W, proj_b):
    effe_i = _sc_gather_f(items, F_feat)
    Tu_pairs = _tc_pairs(jnp.transpose(Tu_weight))
    zidx = (lax.shift_right_logical(users, 13) * (_PLANES // 2)
            + (users & (_PLANES // 2 - 1)))
    zidx, effe_i = lax.optimization_barrier((zidx, effe_i))
    theta2 = _sc_gather_tu(zidx, Tu_pairs)
    gammaT_u, gammaT_i = _tc_gammas(jnp.transpose(gu), jnp.transpose(gi))
    xui2, thetaT, projT = _tc_dense(
        jnp.transpose(gu), jnp.transpose(gi), theta2, users, effe_i,
        proj_W, proj_b)
    return (jnp.reshape(xui2, (B,)), jnp.transpose(gammaT_u),
            jnp.transpose(gammaT_i), jnp.transpose(thetaT),
            jnp.transpose(projT))
